# baseline (speedup 1.0000x reference)
import functools

import jax
import jax.numpy as jnp
from jax import lax
from jax.experimental import pallas as pl
from jax.experimental.pallas import tpu as pltpu

MAX_OUT = 300
MAX_CLS_OUT = 50
SCORE_T = 0.05
NEG = -1e30
B, P, C1, M, H = 4, 5000, 81, 32, 138
HH = H * H


def _nms_kernel(clst_ref, offt_ref, priorst_ref, off_ref, priors_ref,
                coef_ref, boxes_o, coefs_o, scores_o, cls_o, flag_o):
    f32 = jnp.float32
    cls = clst_ref[0]
    mx = jnp.max(cls, axis=0, keepdims=True)
    e = jnp.exp(cls - mx)
    probs = e / jnp.sum(e, axis=0, keepdims=True)

    row = lax.broadcasted_iota(jnp.int32, (C1, P), 0)
    live = (row > 0) & (probs > SCORE_T)
    s0 = jnp.where(live, probs, NEG)
    any_cand = jnp.any(live)

    box_t = offt_ref[0] + priorst_ref[...]
    y1 = box_t[0:1, :]
    x1 = box_t[1:2, :]
    y2 = box_t[2:3, :]
    x2 = box_t[3:4, :]
    a2 = jnp.maximum(y2 - y1, 0.0) * jnp.maximum(x2 - x1, 0.0)

    piota = lax.broadcasted_iota(jnp.int32, (C1, P), 1)
    kiota = lax.broadcasted_iota(jnp.int32, (C1, MAX_CLS_OUT), 1)

    def nms_step(t, carry):
        s, acc_s, acc_i = carry
        m = jnp.max(s, axis=1, keepdims=True)
        cand = jnp.where(s == m, piota, P)
        idx = jnp.min(cand, axis=1, keepdims=True)
        oh = piota == idx
        valid = m > SCORE_T
        by1 = jnp.sum(jnp.where(oh, y1, 0.0), axis=1, keepdims=True)
        bx1 = jnp.sum(jnp.where(oh, x1, 0.0), axis=1, keepdims=True)
        by2 = jnp.sum(jnp.where(oh, y2, 0.0), axis=1, keepdims=True)
        bx2 = jnp.sum(jnp.where(oh, x2, 0.0), axis=1, keepdims=True)
        ba = jnp.maximum(by2 - by1, 0.0) * jnp.maximum(bx2 - bx1, 0.0)
        ih = jnp.maximum(jnp.minimum(by2, y2) - jnp.maximum(by1, y1), 0.0)
        iw = jnp.maximum(jnp.minimum(bx2, x2) - jnp.maximum(bx1, x1), 0.0)
        inter = ih * iw
        union = ba + a2 - inter
        iou = jnp.where(union > 0.0, inter / union, 0.0)
        dec = s * jnp.exp(-(iou * iou))
        dec = jnp.where(oh, NEG, dec)
        dec = jnp.where(dec > SCORE_T, dec, NEG)
        s_next = jnp.where(valid, dec, s)
        acc_s = jnp.where(kiota == t, m, acc_s)
        acc_i = jnp.where(kiota == t, idx.astype(f32), acc_i)
        return s_next, acc_s, acc_i

    zero_ck = jnp.zeros((C1, MAX_CLS_OUT), f32)
    _, sel_s, sel_i = lax.fori_loop(0, MAX_CLS_OUT, nms_step,
                                    (s0, zero_ck, zero_ck))

    valid = sel_s > SCORE_T
    g0 = jnp.where(valid, sel_s, 0.0)
    clsv = jnp.where(valid, row[:, :1].astype(f32), 0.0)

    fio = (lax.broadcasted_iota(jnp.int32, (C1, MAX_CLS_OUT), 0) * MAX_CLS_OUT
           + kiota)
    lio = lax.broadcasted_iota(jnp.int32, (1, MAX_OUT), 1)
    rio = lax.broadcasted_iota(jnp.int32, (MAX_OUT, 1), 0)

    def topk_step(r, carry):
        g, sc_o, cl_o, pi_o = carry
        m = jnp.max(g, axis=(0, 1), keepdims=True)
        fidx = jnp.min(jnp.where(g == m, fio, C1 * MAX_CLS_OUT),
                       axis=(0, 1), keepdims=True)
        oh = fio == fidx
        pidx = jnp.sum(jnp.where(oh, sel_i, 0.0), axis=(0, 1), keepdims=True)
        clr = jnp.sum(jnp.where(oh, clsv, 0.0), axis=(0, 1), keepdims=True)
        sc_o = jnp.where(lio == r, m, sc_o)
        cl_o = jnp.where(lio == r, clr, cl_o)
        pi_o = jnp.where(rio == r, jnp.where(m > 0.0, pidx, -1.0), pi_o)
        g = jnp.where(oh, NEG, g)
        return g, sc_o, cl_o, pi_o

    _, sc_o, cl_o, pi_o = lax.fori_loop(
        0, MAX_OUT, topk_step,
        (g0, jnp.zeros((1, MAX_OUT), f32), jnp.zeros((1, MAX_OUT), f32),
         jnp.full((MAX_OUT, 1), -1.0, f32)))

    onehot = (pi_o.astype(jnp.int32)
              == lax.broadcasted_iota(jnp.int32, (MAX_OUT, P), 1)).astype(f32)
    boxn = off_ref[0] + priors_ref[...]
    hi = jax.lax.Precision.HIGHEST
    boxes_o[0] = jnp.dot(onehot, boxn, precision=hi)
    coefs_o[0] = jnp.dot(onehot, coef_ref[0], precision=hi)
    scores_o[0] = sc_o
    cls_o[0] = cl_o
    flag_o[0] = jnp.full((1, 128), jnp.where(any_cand, 1, 0), jnp.int32)


def _mask_kernel(coef_ref, protot_ref, out_ref):
    hi = jax.lax.Precision.HIGHEST
    mm = jax.nn.sigmoid(jnp.dot(coef_ref[0, 0], protot_ref[0], precision=hi))
    out_ref[0] = mm.reshape(N_TILE, H, H)


N_TILE = 60
N_GRID = 5


@functools.partial(jax.jit, static_argnames=())
def kernel(pred_offset, pred_cls, pred_mask_coef, priors, proto_out):
    f32 = jnp.float32
    cls_t = jnp.transpose(pred_cls, (0, 2, 1))
    off_t = jnp.transpose(pred_offset, (0, 2, 1))
    priors_t = jnp.transpose(priors, (1, 0))
    boxes, coefs, scores3, cls3, flag = pl.pallas_call(
        _nms_kernel,
        grid=(B,),
        compiler_params=pltpu.CompilerParams(
            dimension_semantics=("parallel",)),
        in_specs=[
            pl.BlockSpec((1, C1, P), lambda b: (b, 0, 0)),
            pl.BlockSpec((1, 4, P), lambda b: (b, 0, 0)),
            pl.BlockSpec((4, P), lambda b: (0, 0)),
            pl.BlockSpec((1, P, 4), lambda b: (b, 0, 0)),
            pl.BlockSpec((P, 4), lambda b: (0, 0)),
            pl.BlockSpec((1, P, M), lambda b: (b, 0, 0)),
        ],
        out_specs=[
            pl.BlockSpec((1, MAX_OUT, 4), lambda b: (b, 0, 0)),
            pl.BlockSpec((1, MAX_OUT, M), lambda b: (b, 0, 0)),
            pl.BlockSpec((1, 1, MAX_OUT), lambda b: (b, 0, 0)),
            pl.BlockSpec((1, 1, MAX_OUT), lambda b: (b, 0, 0)),
            pl.BlockSpec((1, 1, 128), lambda b: (b, 0, 0)),
        ],
        out_shape=[
            jax.ShapeDtypeStruct((B, MAX_OUT, 4), f32),
            jax.ShapeDtypeStruct((B, MAX_OUT, M), f32),
            jax.ShapeDtypeStruct((B, 1, MAX_OUT), f32),
            jax.ShapeDtypeStruct((B, 1, MAX_OUT), f32),
            jax.ShapeDtypeStruct((B, 1, 128), jnp.int32),
        ],
    )(cls_t, off_t, priors_t, pred_offset, priors, pred_mask_coef)

    proto_t = jnp.transpose(proto_out.reshape(B, HH, M), (0, 2, 1))
    coefs_t = coefs.reshape(B, N_GRID, N_TILE, M)
    masks = pl.pallas_call(
        _mask_kernel,
        grid=(B, N_GRID),
        compiler_params=pltpu.CompilerParams(
            dimension_semantics=("parallel", "parallel")),
        in_specs=[
            pl.BlockSpec((1, 1, N_TILE, M), lambda b, n: (b, n, 0, 0)),
            pl.BlockSpec((1, M, HH), lambda b, n: (b, 0, 0)),
        ],
        out_specs=pl.BlockSpec((1, N_TILE, H, H), lambda b, n: (b, n, 0, 0)),
        out_shape=jax.ShapeDtypeStruct((B, MAX_OUT, H, H), f32),
    )(coefs_t, proto_t)

    num = flag[:, 0, 0] * jnp.int32(MAX_OUT)
    return (boxes, cls3[:, 0, :], scores3[:, 0, :], masks, num)


# speedup vs baseline: 1.0331x; 1.0331x over previous
import functools

import jax
import jax.numpy as jnp
from jax import lax
from jax.experimental import pallas as pl
from jax.experimental.pallas import tpu as pltpu

MAX_OUT = 300
MAX_CLS_OUT = 50
SCORE_T = 0.05
NEG = -1e30
B, P, C1, M, H = 4, 5000, 81, 32, 138
HH = H * H


def _nms_kernel(clst_ref, offt_ref, priorst_ref, off_ref, priors_ref,
                coef_ref, boxes_o, coefs_o, scores_o, cls_o, flag_o):
    f32 = jnp.float32
    cls = clst_ref[0]
    mx = jnp.max(cls, axis=0, keepdims=True)
    e = jnp.exp(cls - mx)
    probs = e / jnp.sum(e, axis=0, keepdims=True)

    row = lax.broadcasted_iota(jnp.int32, (C1, P), 0)
    live = (row > 0) & (probs > SCORE_T)
    s0 = jnp.where(live, probs, NEG)
    any_cand = jnp.any(live)

    box_t = offt_ref[0] + priorst_ref[...]
    y1 = box_t[0:1, :]
    x1 = box_t[1:2, :]
    y2 = box_t[2:3, :]
    x2 = box_t[3:4, :]
    a2 = jnp.maximum(y2 - y1, 0.0) * jnp.maximum(x2 - x1, 0.0)

    boxn = off_ref[0] + priors_ref[...]
    piota = lax.broadcasted_iota(jnp.int32, (C1, P), 1)
    kiota = lax.broadcasted_iota(jnp.int32, (C1, MAX_CLS_OUT), 1)

    hi = jax.lax.Precision.HIGHEST

    def nms_step(t, carry):
        s, acc_s, acc_i = carry
        m = jnp.max(s, axis=1, keepdims=True)
        cand = jnp.where(s == m, piota, P)
        idx = jnp.min(cand, axis=1, keepdims=True)
        oh = piota == idx
        valid = m > SCORE_T
        sel4 = jnp.dot(oh.astype(f32), boxn, precision=hi)
        by1 = sel4[:, 0:1]
        bx1 = sel4[:, 1:2]
        by2 = sel4[:, 2:3]
        bx2 = sel4[:, 3:4]
        ba = jnp.maximum(by2 - by1, 0.0) * jnp.maximum(bx2 - bx1, 0.0)
        ih = jnp.maximum(jnp.minimum(by2, y2) - jnp.maximum(by1, y1), 0.0)
        iw = jnp.maximum(jnp.minimum(bx2, x2) - jnp.maximum(bx1, x1), 0.0)
        inter = ih * iw
        union = ba + a2 - inter
        iou = jnp.where(union > 0.0, inter / union, 0.0)
        dec = s * jnp.exp(-(iou * iou))
        dec = jnp.where(oh, NEG, dec)
        dec = jnp.where(dec > SCORE_T, dec, NEG)
        s_next = jnp.where(valid, dec, s)
        acc_s = jnp.where(kiota == t, m, acc_s)
        acc_i = jnp.where(kiota == t, idx.astype(f32), acc_i)
        return s_next, acc_s, acc_i

    zero_ck = jnp.zeros((C1, MAX_CLS_OUT), f32)
    _, sel_s, sel_i = lax.fori_loop(0, MAX_CLS_OUT, nms_step,
                                    (s0, zero_ck, zero_ck))

    valid = sel_s > SCORE_T
    g0 = jnp.where(valid, sel_s, 0.0)
    clsv = jnp.where(valid, row[:, :1].astype(f32), 0.0)

    fio = (lax.broadcasted_iota(jnp.int32, (C1, MAX_CLS_OUT), 0) * MAX_CLS_OUT
           + kiota)
    packed = (fio << 13) | sel_i.astype(jnp.int32)
    lio = lax.broadcasted_iota(jnp.int32, (1, MAX_OUT), 1)
    rio = lax.broadcasted_iota(jnp.int32, (MAX_OUT, 1), 0)
    imax = jnp.int32(0x7FFFFFFF)

    def topk_step(r, carry):
        g, sc_o, cl_o, pi_o = carry
        m = jnp.max(g, axis=(0, 1), keepdims=True)
        key = jnp.min(jnp.where(g == m, packed, imax),
                      axis=(0, 1), keepdims=True)
        fidx = key >> 13
        pidx = (key & jnp.int32(0x1FFF)).astype(f32)
        clr = (fidx // MAX_CLS_OUT).astype(f32)
        ok = m > 0.0
        oh = fio == fidx
        sc_o = jnp.where(lio == r, m, sc_o)
        cl_o = jnp.where(lio == r, jnp.where(ok, clr, 0.0), cl_o)
        pi_o = jnp.where(rio == r, jnp.where(ok, pidx, -1.0), pi_o)
        g = jnp.where(oh, NEG, g)
        return g, sc_o, cl_o, pi_o

    _, sc_o, cl_o, pi_o = lax.fori_loop(
        0, MAX_OUT, topk_step,
        (g0, jnp.zeros((1, MAX_OUT), f32), jnp.zeros((1, MAX_OUT), f32),
         jnp.full((MAX_OUT, 1), -1.0, f32)))

    onehot = (pi_o.astype(jnp.int32)
              == lax.broadcasted_iota(jnp.int32, (MAX_OUT, P), 1)).astype(f32)
    boxes_o[0] = jnp.dot(onehot, boxn, precision=hi)
    coefs_o[0] = jnp.dot(onehot, coef_ref[0], precision=hi)
    scores_o[0] = sc_o
    cls_o[0] = cl_o
    flag_o[0] = jnp.full((1, 128), jnp.where(any_cand, 1, 0), jnp.int32)


def _mask_kernel(coef_ref, protot_ref, out_ref):
    hi = jax.lax.Precision.HIGHEST
    mm = jax.nn.sigmoid(jnp.dot(coef_ref[0, 0], protot_ref[0], precision=hi))
    out_ref[0] = mm.reshape(N_TILE, H, H)


N_TILE = 60
N_GRID = 5


@functools.partial(jax.jit, static_argnames=())
def kernel(pred_offset, pred_cls, pred_mask_coef, priors, proto_out):
    f32 = jnp.float32
    cls_t = jnp.transpose(pred_cls, (0, 2, 1))
    off_t = jnp.transpose(pred_offset, (0, 2, 1))
    priors_t = jnp.transpose(priors, (1, 0))
    boxes, coefs, scores3, cls3, flag = pl.pallas_call(
        _nms_kernel,
        grid=(B,),
        compiler_params=pltpu.CompilerParams(
            dimension_semantics=("parallel",)),
        in_specs=[
            pl.BlockSpec((1, C1, P), lambda b: (b, 0, 0)),
            pl.BlockSpec((1, 4, P), lambda b: (b, 0, 0)),
            pl.BlockSpec((4, P), lambda b: (0, 0)),
            pl.BlockSpec((1, P, 4), lambda b: (b, 0, 0)),
            pl.BlockSpec((P, 4), lambda b: (0, 0)),
            pl.BlockSpec((1, P, M), lambda b: (b, 0, 0)),
        ],
        out_specs=[
            pl.BlockSpec((1, MAX_OUT, 4), lambda b: (b, 0, 0)),
            pl.BlockSpec((1, MAX_OUT, M), lambda b: (b, 0, 0)),
            pl.BlockSpec((1, 1, MAX_OUT), lambda b: (b, 0, 0)),
            pl.BlockSpec((1, 1, MAX_OUT), lambda b: (b, 0, 0)),
            pl.BlockSpec((1, 1, 128), lambda b: (b, 0, 0)),
        ],
        out_shape=[
            jax.ShapeDtypeStruct((B, MAX_OUT, 4), f32),
            jax.ShapeDtypeStruct((B, MAX_OUT, M), f32),
            jax.ShapeDtypeStruct((B, 1, MAX_OUT), f32),
            jax.ShapeDtypeStruct((B, 1, MAX_OUT), f32),
            jax.ShapeDtypeStruct((B, 1, 128), jnp.int32),
        ],
    )(cls_t, off_t, priors_t, pred_offset, priors, pred_mask_coef)

    proto_t = jnp.transpose(proto_out.reshape(B, HH, M), (0, 2, 1))
    coefs_t = coefs.reshape(B, N_GRID, N_TILE, M)
    masks = pl.pallas_call(
        _mask_kernel,
        grid=(B, N_GRID),
        compiler_params=pltpu.CompilerParams(
            dimension_semantics=("parallel", "parallel")),
        in_specs=[
            pl.BlockSpec((1, 1, N_TILE, M), lambda b, n: (b, n, 0, 0)),
            pl.BlockSpec((1, M, HH), lambda b, n: (b, 0, 0)),
        ],
        out_specs=pl.BlockSpec((1, N_TILE, H, H), lambda b, n: (b, n, 0, 0)),
        out_shape=jax.ShapeDtypeStruct((B, MAX_OUT, H, H), f32),
    )(coefs_t, proto_t)

    num = flag[:, 0, 0] * jnp.int32(MAX_OUT)
    return (boxes, cls3[:, 0, :], scores3[:, 0, :], masks, num)
